# baseline (speedup 1.0000x reference)
import functools

import jax
import jax.numpy as jnp
from jax import lax
from jax.experimental import pallas as pl
from jax.experimental.pallas import tpu as pltpu
from jax.experimental.pallas import tpu_sc as plsc

_BK = 512
_NC, _NS = 2, 16
_NW = _NC * _NS
_P_PAD = 2048


def _tc_body(num_cols, x_ref, t_ref, hf_ref, hc_ref, bs_ref,
             hf_acc, hc_acc):
    k = pl.program_id(0)
    nk = pl.num_programs(0)
    ntiles = hf_ref.shape[0]
    lane_pad = hf_acc.shape[1]
    x = x_ref[...]
    t = t_ref[...].astype(jnp.float32)
    sig = 1.0 / (1.0 + jnp.exp(-x))
    bce = jnp.maximum(x, 0.0) - x * t - jnp.log(jnp.maximum(sig, 1.0 - sig))
    bf = jnp.bfloat16
    f1 = (sig * (1.0 - t)).astype(bf)
    f3 = sig.astype(bf)
    f2 = (sig * (1.0 - t) * bce).astype(bf)
    g1 = (1.0 - sig).astype(bf)
    g2 = ((1.0 - sig) * t).astype(bf)
    g3 = ((1.0 - sig) * t * bce).astype(bf)

    def dot1(a, b):
        return lax.dot_general(a, b, (((1,), (1,)), ((), ())),
                               preferred_element_type=jnp.float32)

    hf = dot1(f1, g1) + dot1(f3, g2)
    hc = dot1(f2, g1) + dot1(f3, g3)
    bs = jnp.sum(bce)

    @pl.when(k == 0)
    def _():
        if lane_pad > num_cols:
            zpad = jnp.zeros((num_cols, lane_pad - num_cols), jnp.float32)
            hf_acc[:, num_cols:lane_pad] = zpad
            hc_acc[:, num_cols:lane_pad] = zpad
        hf_acc[:, 0:num_cols] = hf
        hc_acc[:, 0:num_cols] = hc
        bs_ref[0, 0] = bs

    @pl.when(k > 0)
    def _():
        hf_acc[:, 0:num_cols] += hf
        hc_acc[:, 0:num_cols] += hc
        bs_ref[0, 0] += bs

    @pl.when(k == nk - 1)
    def _():
        for j in range(ntiles):
            lo, hi = j * 128, (j + 1) * 128
            hf_ref[j] = hf_acc[:, lo:hi]
            hc_ref[j] = hc_acc[:, lo:hi]


def _make_tc_call(batch, num_cols):
    grid = batch // _BK
    ntiles = (num_cols + 127) // 128
    lane_pad = ntiles * 128
    return pl.pallas_call(
        functools.partial(_tc_body, num_cols),
        grid=(grid,),
        in_specs=[
            pl.BlockSpec((num_cols, _BK), lambda k: (0, k)),
            pl.BlockSpec((num_cols, _BK), lambda k: (0, k)),
        ],
        out_specs=[
            pl.BlockSpec((ntiles, num_cols, 128), lambda k: (0, 0, 0)),
            pl.BlockSpec((ntiles, num_cols, 128), lambda k: (0, 0, 0)),
            pl.BlockSpec((1, 1), lambda k: (0, 0),
                         memory_space=pltpu.SMEM),
        ],
        out_shape=[
            jax.ShapeDtypeStruct((ntiles, num_cols, 128), jnp.float32),
            jax.ShapeDtypeStruct((ntiles, num_cols, 128), jnp.float32),
            jax.ShapeDtypeStruct((1, 1), jnp.float32),
        ],
        scratch_shapes=[
            pltpu.VMEM((num_cols, lane_pad), jnp.float32),
            pltpu.VMEM((num_cols, lane_pad), jnp.float32),
        ],
        compiler_params=pltpu.CompilerParams(
            dimension_semantics=("arbitrary",)),
    )


def _sc_body(num_pairs, num_cols, fl_ref, fr_ref, hf_ref, hc_ref,
             outf_ref, outc_ref,
             idxl_v, idxr_v, flat_v, valsf_v, valsc_v, accf_v, accc_v, sem):
    wid = lax.axis_index("s") * _NC + lax.axis_index("c")
    per_w = _P_PAD // _NW
    base = wid * per_w
    pltpu.sync_copy(fl_ref.at[pl.ds(base, per_w)], idxl_v)
    pltpu.sync_copy(fr_ref.at[pl.ds(base, per_w)], idxr_v)
    for sub in range(per_w // 16):
        l = idxl_v[pl.ds(sub * 16, 16)]
        r = idxr_v[pl.ds(sub * 16, 16)]
        jt = lax.shift_right_logical(r, 7)
        c = jnp.bitwise_and(r, 127)
        flat_v[pl.ds(sub * 16, 16)] = jt * (num_cols * 128) + l * 128 + c
    pltpu.async_copy(hf_ref.at[flat_v], valsf_v, sem).wait()
    pltpu.async_copy(hc_ref.at[flat_v], valsc_v, sem).wait()
    accf = jnp.zeros((16,), jnp.float32)
    accc = jnp.zeros((16,), jnp.float32)
    for sub in range(per_w // 16):
        ids = base + sub * 16 + lax.iota(jnp.int32, 16)
        valid = ids < num_pairs
        vf = jnp.where(valid, valsf_v[pl.ds(sub * 16, 16)], 0.0)
        vc = jnp.where(valid, valsc_v[pl.ds(sub * 16, 16)], 0.0)
        accf = accf + vf
        accc = accc + vc
    accf_v[...] = accf
    accc_v[...] = accc
    pltpu.sync_copy(accf_v, outf_ref.at[wid])
    pltpu.sync_copy(accc_v, outc_ref.at[wid])


@functools.lru_cache(maxsize=2)
def _make_sc_call(num_pairs, num_cols):
    per_w = _P_PAD // _NW
    return pl.kernel(
        functools.partial(_sc_body, num_pairs, num_cols),
        out_type=(
            jax.ShapeDtypeStruct((_NW, 16), jnp.float32),
            jax.ShapeDtypeStruct((_NW, 16), jnp.float32),
        ),
        mesh=plsc.VectorSubcoreMesh(core_axis_name="c", subcore_axis_name="s"),
        scratch_types=[
            pltpu.VMEM((per_w,), jnp.int32),
            pltpu.VMEM((per_w,), jnp.int32),
            pltpu.VMEM((per_w,), jnp.int32),
            pltpu.VMEM((per_w,), jnp.float32),
            pltpu.VMEM((per_w,), jnp.float32),
            pltpu.VMEM((16,), jnp.float32),
            pltpu.VMEM((16,), jnp.float32),
            pltpu.SemaphoreType.DMA,
        ],
    )


def kernel(input, target, filter_l, filter_r):
    batch, num_cols = input.shape
    num_pairs = filter_l.shape[0]
    hf, hc, bs = _make_tc_call(batch, num_cols)(input.T, target.T)
    pad = jnp.zeros((_P_PAD - num_pairs,), jnp.int32)
    flp = jnp.concatenate([filter_l, pad])
    frp = jnp.concatenate([filter_r, pad])
    outf, outc = _make_sc_call(num_pairs, num_cols)(
        flp, frp, hf.reshape(-1), hc.reshape(-1))
    denom = batch * num_cols
    base_mean = bs[0, 0] / denom
    fuzzy = outf.sum() / denom
    cross = outc.sum() / denom
    total = base_mean + 0.01 * cross
    return (total, base_mean, fuzzy, 0.1 * fuzzy)


# speedup vs baseline: 1.0619x; 1.0619x over previous
import functools

import jax
import jax.numpy as jnp
from jax import lax
from jax.experimental import pallas as pl
from jax.experimental.pallas import tpu as pltpu
from jax.experimental.pallas import tpu_sc as plsc

_BK = 1024
_NC, _NS = 2, 16
_NW = _NC * _NS
_P_PAD = 2048


def _tc_body(num_cols, x_ref, t_ref, hf_ref, hc_ref, bs_ref):
    k = pl.program_id(0)
    ntiles = hf_ref.shape[0]
    x = x_ref[...]
    t = t_ref[...].astype(jnp.float32)
    sig = 1.0 / (1.0 + jnp.exp(-x))
    bce = jnp.maximum(x, 0.0) - x * t - jnp.log(jnp.maximum(sig, 1.0 - sig))
    bf = jnp.bfloat16
    f1 = (sig * (1.0 - t)).astype(bf)
    f3 = sig.astype(bf)
    f2 = (sig * (1.0 - t) * bce).astype(bf)
    g1 = (1.0 - sig).astype(bf)
    g2 = ((1.0 - sig) * t).astype(bf)
    g3 = ((1.0 - sig) * t * bce).astype(bf)

    def dot1(a, b):
        return lax.dot_general(a, b, (((1,), (1,)), ((), ())),
                               preferred_element_type=jnp.float32)

    hf = dot1(f1, g1) + dot1(f3, g2)
    hc = dot1(f2, g1) + dot1(f3, g3)
    bs = jnp.sum(bce)

    def tile(h, j):
        lo, hi = j * 128, (j + 1) * 128
        if hi <= num_cols:
            return h[:, lo:hi]
        pad = jnp.zeros((h.shape[0], hi - num_cols), jnp.float32)
        return jnp.concatenate([h[:, lo:num_cols], pad], axis=1)

    @pl.when(k == 0)
    def _():
        for j in range(ntiles):
            hf_ref[j] = tile(hf, j)
            hc_ref[j] = tile(hc, j)
        bs_ref[0, 0] = bs

    @pl.when(k > 0)
    def _():
        for j in range(ntiles):
            hf_ref[j] += tile(hf, j)
            hc_ref[j] += tile(hc, j)
        bs_ref[0, 0] += bs


def _make_tc_call(batch, num_cols):
    grid = batch // _BK
    ntiles = (num_cols + 127) // 128
    lane_pad = ntiles * 128
    return pl.pallas_call(
        functools.partial(_tc_body, num_cols),
        grid=(grid,),
        in_specs=[
            pl.BlockSpec((num_cols, _BK), lambda k: (0, k)),
            pl.BlockSpec((num_cols, _BK), lambda k: (0, k)),
        ],
        out_specs=[
            pl.BlockSpec((ntiles, num_cols, 128), lambda k: (0, 0, 0)),
            pl.BlockSpec((ntiles, num_cols, 128), lambda k: (0, 0, 0)),
            pl.BlockSpec((1, 1), lambda k: (0, 0),
                         memory_space=pltpu.SMEM),
        ],
        out_shape=[
            jax.ShapeDtypeStruct((ntiles, num_cols, 128), jnp.float32),
            jax.ShapeDtypeStruct((ntiles, num_cols, 128), jnp.float32),
            jax.ShapeDtypeStruct((1, 1), jnp.float32),
        ],
        compiler_params=pltpu.CompilerParams(
            dimension_semantics=("arbitrary",)),
    )


def _sc_body(num_pairs, num_cols, fl_ref, fr_ref, hf_ref, hc_ref,
             outf_ref, outc_ref,
             idxl_v, idxr_v, flat_v, valsf_v, valsc_v, accf_v, accc_v, sem):
    wid = lax.axis_index("s") * _NC + lax.axis_index("c")
    per_w = _P_PAD // _NW
    base = wid * per_w
    pltpu.sync_copy(fl_ref.at[pl.ds(base, per_w)], idxl_v)
    pltpu.sync_copy(fr_ref.at[pl.ds(base, per_w)], idxr_v)
    for sub in range(per_w // 16):
        l = idxl_v[pl.ds(sub * 16, 16)]
        r = idxr_v[pl.ds(sub * 16, 16)]
        jt = lax.shift_right_logical(r, 7)
        c = jnp.bitwise_and(r, 127)
        flat_v[pl.ds(sub * 16, 16)] = jt * (num_cols * 128) + l * 128 + c
    pltpu.async_copy(hf_ref.at[flat_v], valsf_v, sem).wait()
    pltpu.async_copy(hc_ref.at[flat_v], valsc_v, sem).wait()
    accf = jnp.zeros((16,), jnp.float32)
    accc = jnp.zeros((16,), jnp.float32)
    for sub in range(per_w // 16):
        ids = base + sub * 16 + lax.iota(jnp.int32, 16)
        valid = ids < num_pairs
        vf = jnp.where(valid, valsf_v[pl.ds(sub * 16, 16)], 0.0)
        vc = jnp.where(valid, valsc_v[pl.ds(sub * 16, 16)], 0.0)
        accf = accf + vf
        accc = accc + vc
    accf_v[...] = accf
    accc_v[...] = accc
    pltpu.sync_copy(accf_v, outf_ref.at[wid])
    pltpu.sync_copy(accc_v, outc_ref.at[wid])


@functools.lru_cache(maxsize=2)
def _make_sc_call(num_pairs, num_cols):
    per_w = _P_PAD // _NW
    return pl.kernel(
        functools.partial(_sc_body, num_pairs, num_cols),
        out_type=(
            jax.ShapeDtypeStruct((_NW, 16), jnp.float32),
            jax.ShapeDtypeStruct((_NW, 16), jnp.float32),
        ),
        mesh=plsc.VectorSubcoreMesh(core_axis_name="c", subcore_axis_name="s"),
        scratch_types=[
            pltpu.VMEM((per_w,), jnp.int32),
            pltpu.VMEM((per_w,), jnp.int32),
            pltpu.VMEM((per_w,), jnp.int32),
            pltpu.VMEM((per_w,), jnp.float32),
            pltpu.VMEM((per_w,), jnp.float32),
            pltpu.VMEM((16,), jnp.float32),
            pltpu.VMEM((16,), jnp.float32),
            pltpu.SemaphoreType.DMA,
        ],
    )


def kernel(input, target, filter_l, filter_r):
    batch, num_cols = input.shape
    num_pairs = filter_l.shape[0]
    hf, hc, bs = _make_tc_call(batch, num_cols)(input.T, target.T)
    pad = jnp.zeros((_P_PAD - num_pairs,), jnp.int32)
    flp = jnp.concatenate([filter_l, pad])
    frp = jnp.concatenate([filter_r, pad])
    outf, outc = _make_sc_call(num_pairs, num_cols)(
        flp, frp, hf.reshape(-1), hc.reshape(-1))
    denom = batch * num_cols
    base_mean = bs[0, 0] / denom
    fuzzy = outf.sum() / denom
    cross = outc.sum() / denom
    total = base_mean + 0.01 * cross
    return (total, base_mean, fuzzy, 0.1 * fuzzy)


# speedup vs baseline: 1.1163x; 1.0512x over previous
import functools

import jax
import jax.numpy as jnp
from jax import lax
from jax.experimental import pallas as pl
from jax.experimental.pallas import tpu as pltpu
from jax.experimental.pallas import tpu_sc as plsc

_BK = 1024
_NC, _NS = 2, 16
_NW = _NC * _NS
_P_PAD = 2048


def _tc_body(num_cols, x_ref, t_ref, hf_ref, hc_ref, bs_ref):
    k = pl.program_id(0)
    ntiles = hf_ref.shape[0]
    x = x_ref[...]
    t = t_ref[...].astype(jnp.float32)
    sig = 1.0 / (1.0 + jnp.exp(-x))
    bce = jnp.maximum(x, 0.0) - x * t - jnp.log(jnp.maximum(sig, 1.0 - sig))
    bf = jnp.bfloat16
    f1 = (sig * (1.0 - t)).astype(bf)
    f3 = sig.astype(bf)
    f2 = (sig * (1.0 - t) * bce).astype(bf)
    g1 = (1.0 - sig).astype(bf)
    g2 = ((1.0 - sig) * t).astype(bf)
    g3 = ((1.0 - sig) * t * bce).astype(bf)

    def dot1(a, b):
        return lax.dot_general(a, b, (((1,), (1,)), ((), ())),
                               preferred_element_type=jnp.float32)

    hf = dot1(f1, g1) + dot1(f3, g2)
    hc = dot1(f2, g1) + dot1(f3, g3)
    bs = jnp.sum(bce)

    def tile(h, j):
        lo, hi = j * 128, (j + 1) * 128
        if hi <= num_cols:
            return h[:, lo:hi]
        pad = jnp.zeros((h.shape[0], hi - num_cols), jnp.float32)
        return jnp.concatenate([h[:, lo:num_cols], pad], axis=1)

    @pl.when(k == 0)
    def _():
        for j in range(ntiles):
            hf_ref[j] = tile(hf, j)
            hc_ref[j] = tile(hc, j)
        bs_ref[0, 0] = bs

    @pl.when(k > 0)
    def _():
        for j in range(ntiles):
            hf_ref[j] += tile(hf, j)
            hc_ref[j] += tile(hc, j)
        bs_ref[0, 0] += bs


def _make_tc_call(batch, num_cols):
    grid = batch // _BK
    ntiles = (num_cols + 127) // 128
    lane_pad = ntiles * 128
    return pl.pallas_call(
        functools.partial(_tc_body, num_cols),
        grid=(grid,),
        in_specs=[
            pl.BlockSpec((num_cols, _BK), lambda k: (0, k)),
            pl.BlockSpec((num_cols, _BK), lambda k: (0, k)),
        ],
        out_specs=[
            pl.BlockSpec((ntiles, num_cols, 128), lambda k: (0, 0, 0)),
            pl.BlockSpec((ntiles, num_cols, 128), lambda k: (0, 0, 0)),
            pl.BlockSpec((1, 1), lambda k: (0, 0),
                         memory_space=pltpu.SMEM),
        ],
        out_shape=[
            jax.ShapeDtypeStruct((ntiles, num_cols, 128), jnp.float32),
            jax.ShapeDtypeStruct((ntiles, num_cols, 128), jnp.float32),
            jax.ShapeDtypeStruct((1, 1), jnp.float32),
        ],
        compiler_params=pltpu.CompilerParams(
            dimension_semantics=("arbitrary",)),
    )


def _sc_body(num_pairs, num_cols, fl_ref, fr_ref, hf_ref, hc_ref, out_ref,
             idxl_v, idxr_v, flat_v, valsf_v, valsc_v, accf_v, accc_v, sem):
    wid = lax.axis_index("s") * _NC + lax.axis_index("c")
    per_w = _P_PAD // _NW
    base = jnp.minimum(wid * per_w, num_pairs - per_w)
    pltpu.sync_copy(fl_ref.at[pl.ds(base, per_w)], idxl_v)
    pltpu.sync_copy(fr_ref.at[pl.ds(base, per_w)], idxr_v)
    for sub in range(per_w // 16):
        l = idxl_v[pl.ds(sub * 16, 16)]
        r = idxr_v[pl.ds(sub * 16, 16)]
        jt = lax.shift_right_logical(r, 7)
        c = jnp.bitwise_and(r, 127)
        flat_v[pl.ds(sub * 16, 16)] = jt * (num_cols * 128) + l * 128 + c
    pltpu.async_copy(hf_ref.at[flat_v], valsf_v, sem).wait()
    pltpu.async_copy(hc_ref.at[flat_v], valsc_v, sem).wait()
    accf = jnp.zeros((16,), jnp.float32)
    accc = jnp.zeros((16,), jnp.float32)
    lo = wid * per_w
    hi = jnp.minimum(lo + per_w, num_pairs)
    for sub in range(per_w // 16):
        ids = base + sub * 16 + lax.iota(jnp.int32, 16)
        valid = (ids >= lo) & (ids < hi)
        vf = jnp.where(valid, valsf_v[pl.ds(sub * 16, 16)], 0.0)
        vc = jnp.where(valid, valsc_v[pl.ds(sub * 16, 16)], 0.0)
        accf = accf + vf
        accc = accc + vc
    accf_v[...] = accf
    accc_v[...] = accc
    pltpu.sync_copy(accf_v, out_ref.at[0, wid])
    pltpu.sync_copy(accc_v, out_ref.at[1, wid])


@functools.lru_cache(maxsize=2)
def _make_sc_call(num_pairs, num_cols):
    per_w = _P_PAD // _NW
    return pl.kernel(
        functools.partial(_sc_body, num_pairs, num_cols),
        out_type=jax.ShapeDtypeStruct((2, _NW, 16), jnp.float32),
        mesh=plsc.VectorSubcoreMesh(core_axis_name="c", subcore_axis_name="s"),
        scratch_types=[
            pltpu.VMEM((per_w,), jnp.int32),
            pltpu.VMEM((per_w,), jnp.int32),
            pltpu.VMEM((per_w,), jnp.int32),
            pltpu.VMEM((per_w,), jnp.float32),
            pltpu.VMEM((per_w,), jnp.float32),
            pltpu.VMEM((16,), jnp.float32),
            pltpu.VMEM((16,), jnp.float32),
            pltpu.SemaphoreType.DMA,
        ],
    )


def kernel(input, target, filter_l, filter_r):
    batch, num_cols = input.shape
    num_pairs = filter_l.shape[0]
    hf, hc, bs = _make_tc_call(batch, num_cols)(input.T, target.T)
    out = _make_sc_call(num_pairs, num_cols)(
        filter_l, filter_r, hf.reshape(-1), hc.reshape(-1))
    denom = batch * num_cols
    sums = out.sum(axis=(1, 2))
    base_mean = bs[0, 0] / denom
    fuzzy = sums[0] / denom
    cross = sums[1] / denom
    total = base_mean + 0.01 * cross
    return (total, base_mean, fuzzy, 0.1 * fuzzy)


# speedup vs baseline: 1.1206x; 1.0039x over previous
import functools

import jax
import jax.numpy as jnp
from jax import lax
from jax.experimental import pallas as pl
from jax.experimental.pallas import tpu as pltpu
from jax.experimental.pallas import tpu_sc as plsc

_BK = 1024
_NC, _NS = 2, 16
_NW = _NC * _NS
_P_PAD = 2048


def _tc_body(num_cols, x_ref, t_ref, hf_ref, hc_ref, bs_ref):
    k = pl.program_id(0)
    ntiles = hf_ref.shape[0]
    x = x_ref[...]
    t = t_ref[...].astype(jnp.float32)
    sig = 1.0 / (1.0 + jnp.exp(-x))
    bce = jnp.maximum(x, 0.0) - x * t - jnp.log(jnp.maximum(sig, 1.0 - sig))
    bf = jnp.bfloat16
    f1 = (sig * (1.0 - t)).astype(bf)
    f3 = sig.astype(bf)
    f2 = (sig * (1.0 - t) * bce).astype(bf)
    g1 = (1.0 - sig).astype(bf)
    g2 = ((1.0 - sig) * t).astype(bf)
    g3 = ((1.0 - sig) * t * bce).astype(bf)

    def dot1(a, b):
        return lax.dot_general(a, b, (((1,), (1,)), ((), ())),
                               preferred_element_type=jnp.float32)

    hf = dot1(f1, g1) + dot1(f3, g2)
    hc = dot1(f2, g1) + dot1(f3, g3)
    bs = jnp.sum(bce)

    def tile(h, j):
        lo, hi = j * 128, (j + 1) * 128
        if hi <= num_cols:
            return h[:, lo:hi]
        pad = jnp.zeros((h.shape[0], hi - num_cols), jnp.float32)
        return jnp.concatenate([h[:, lo:num_cols], pad], axis=1)

    @pl.when(k == 0)
    def _():
        for j in range(ntiles):
            hf_ref[j] = tile(hf, j)
            hc_ref[j] = tile(hc, j)
        bs_ref[0, 0] = bs

    @pl.when(k > 0)
    def _():
        for j in range(ntiles):
            hf_ref[j] += tile(hf, j)
            hc_ref[j] += tile(hc, j)
        bs_ref[0, 0] += bs


def _make_tc_call(batch, num_cols):
    grid = batch // _BK
    ntiles = (num_cols + 127) // 128
    return pl.pallas_call(
        functools.partial(_tc_body, num_cols),
        grid=(grid,),
        in_specs=[
            pl.BlockSpec((num_cols, _BK), lambda k: (0, k)),
            pl.BlockSpec((num_cols, _BK), lambda k: (0, k)),
        ],
        out_specs=[
            pl.BlockSpec((ntiles, num_cols, 128), lambda k: (0, 0, 0)),
            pl.BlockSpec((ntiles, num_cols, 128), lambda k: (0, 0, 0)),
            pl.BlockSpec((1, 1), lambda k: (0, 0),
                         memory_space=pltpu.SMEM),
        ],
        out_shape=[
            jax.ShapeDtypeStruct((ntiles, num_cols, 128), jnp.float32),
            jax.ShapeDtypeStruct((ntiles, num_cols, 128), jnp.float32),
            jax.ShapeDtypeStruct((1, 1), jnp.float32),
        ],
        compiler_params=pltpu.CompilerParams(
            dimension_semantics=("arbitrary",)),
    )


def _sc_body(num_pairs, num_cols, fl_ref, fr_ref, hf_ref, hc_ref, out_ref,
             idxl_v, idxr_v, flat_v, valsf_v, valsc_v, accf_v, accc_v, sem):
    wid = lax.axis_index("s") * _NC + lax.axis_index("c")
    per_w = _P_PAD // _NW
    base = jnp.minimum(wid * per_w, num_pairs - per_w)
    pltpu.sync_copy(fl_ref.at[pl.ds(base, per_w)], idxl_v)
    pltpu.sync_copy(fr_ref.at[pl.ds(base, per_w)], idxr_v)
    for sub in range(per_w // 16):
        l = idxl_v[pl.ds(sub * 16, 16)]
        r = idxr_v[pl.ds(sub * 16, 16)]
        jt = lax.shift_right_logical(r, 7)
        c = jnp.bitwise_and(r, 127)
        flat_v[pl.ds(sub * 16, 16)] = jt * (num_cols * 128) + l * 128 + c
    pltpu.async_copy(hf_ref.at[flat_v], valsf_v, sem).wait()
    pltpu.async_copy(hc_ref.at[flat_v], valsc_v, sem).wait()
    accf = jnp.zeros((16,), jnp.float32)
    accc = jnp.zeros((16,), jnp.float32)
    lo = wid * per_w
    hi = jnp.minimum(lo + per_w, num_pairs)
    for sub in range(per_w // 16):
        ids = base + sub * 16 + lax.iota(jnp.int32, 16)
        valid = (ids >= lo) & (ids < hi)
        vf = jnp.where(valid, valsf_v[pl.ds(sub * 16, 16)], 0.0)
        vc = jnp.where(valid, valsc_v[pl.ds(sub * 16, 16)], 0.0)
        accf = accf + vf
        accc = accc + vc
    accf_v[...] = accf
    accc_v[...] = accc
    pltpu.sync_copy(accf_v, out_ref.at[0, wid])
    pltpu.sync_copy(accc_v, out_ref.at[1, wid])


@functools.lru_cache(maxsize=2)
def _make_sc_call(num_pairs, num_cols):
    per_w = _P_PAD // _NW
    return pl.kernel(
        functools.partial(_sc_body, num_pairs, num_cols),
        out_type=jax.ShapeDtypeStruct((2, _NW, 16), jnp.float32),
        mesh=plsc.VectorSubcoreMesh(core_axis_name="c", subcore_axis_name="s"),
        scratch_types=[
            pltpu.VMEM((per_w,), jnp.int32),
            pltpu.VMEM((per_w,), jnp.int32),
            pltpu.VMEM((per_w,), jnp.int32),
            pltpu.VMEM((per_w,), jnp.float32),
            pltpu.VMEM((per_w,), jnp.float32),
            pltpu.VMEM((16,), jnp.float32),
            pltpu.VMEM((16,), jnp.float32),
            pltpu.SemaphoreType.DMA,
        ],
    )


def kernel(input, target, filter_l, filter_r):
    batch, num_cols = input.shape
    num_pairs = filter_l.shape[0]
    hf, hc, bs = _make_tc_call(batch, num_cols)(input.T, target.T)
    out = _make_sc_call(num_pairs, num_cols)(
        filter_l, filter_r, hf.reshape(-1), hc.reshape(-1))
    denom = batch * num_cols
    sums = out.sum(axis=(1, 2))
    base_mean = bs[0, 0] / denom
    fuzzy = sums[0] / denom
    cross = sums[1] / denom
    total = base_mean + 0.01 * cross
    return (total, base_mean, fuzzy, 0.1 * fuzzy)
